# baseline (speedup 1.0000x reference)
import functools

import jax
import jax.numpy as jnp
from jax import lax
from jax.experimental import pallas as pl
from jax.experimental.pallas import tpu as pltpu
from jax.experimental.pallas import tpu_sc as plsc

NC = 2
NS = 16
NW = NC * NS

EMB_D = 64
GRP = 128
G_PER_IT = 8
ROWS_PER_IT = GRP * G_PER_IT


def _gather_kernel(n_groups_per_worker: int, n_iters: int):
    mesh = plsc.VectorSubcoreMesh(core_axis_name="c", subcore_axis_name="s",
                                  num_cores=NC, num_subcores=NS)
    total_rows = NW * n_groups_per_worker * GRP

    @functools.partial(
        pl.kernel,
        out_type=jax.ShapeDtypeStruct((total_rows, EMB_D), jnp.float32),
        mesh=mesh,
        scratch_types=[
            pltpu.VMEM((n_groups_per_worker, GRP), jnp.int32),
            pltpu.VMEM((ROWS_PER_IT, EMB_D), jnp.float32),
            pltpu.SemaphoreType.DMA,
        ],
        compiler_params=pltpu.CompilerParams(use_tc_tiling_on_sc=False),
    )
    def body(ids_hbm, table_hbm, out_hbm, idx_v, rows_v, gsem):
        wid = lax.axis_index("s") * NC + lax.axis_index("c")
        out_base = wid * (n_groups_per_worker * GRP)
        pltpu.sync_copy(ids_hbm.at[wid], idx_v)

        @pl.loop(0, n_iters)
        def _(it):
            copies = []
            for g in range(G_PER_IT):
                copies.append(pltpu.async_copy(
                    table_hbm.at[idx_v.at[it * G_PER_IT + g]],
                    rows_v.at[pl.ds(g * GRP, GRP)],
                    gsem,
                ))
            for c in copies:
                c.wait()
            pltpu.sync_copy(
                rows_v,
                out_hbm.at[pl.ds(out_base + it * ROWS_PER_IT, ROWS_PER_IT)],
            )

    return body


def kernel(codec_ids, table):
    batch, seq = codec_ids.shape
    n = batch * seq
    assert n % (NW * GRP) == 0
    n_groups_per_worker = n // (NW * GRP)
    assert n_groups_per_worker % G_PER_IT == 0
    n_iters = n_groups_per_worker // G_PER_IT

    ids = codec_ids.astype(jnp.int32).reshape(NW, n_groups_per_worker, GRP)
    out = _gather_kernel(n_groups_per_worker, n_iters)(ids, table)
    return out.reshape(batch, seq, EMB_D)
